# baseline (speedup 1.0000x reference)
import functools

import jax
import jax.numpy as jnp
from jax import lax
from jax.experimental import pallas as pl
from jax.experimental.pallas import tpu as pltpu
from jax.experimental.pallas import tpu_sc as plsc

E = 8
D = 768
T = 8192
BT = 256
NBLK = T // BT + E
TPAD = NBLK * BT
NBEXP = 48
TA = 1024
GA = T // TA
TB = 256
GB = T // TB
NC, NS = 2, 16
NW = NC * NS
TPW = T // NW
CH = 128
NCH = TPW // CH
CS = 64
NCS = TPW // CS


def _gate_body(x_ref, gw_ref, sm_ref, eid_ref, w_ref, cnt_ref):
    x = x_ref[...]
    logits = lax.dot_general(x, gw_ref[...], (((1,), (1,)), ((), ())),
                             preferred_element_type=jnp.float32)
    m = jnp.max(logits, axis=1, keepdims=True)
    ex = jnp.exp(logits - m)
    sm = ex / jnp.sum(ex, axis=1, keepdims=True)
    sm_ref[...] = sm
    mx = jnp.max(sm, axis=1, keepdims=True)
    w_ref[...] = mx
    lanes = lax.broadcasted_iota(jnp.int32, (TA, E), 1)
    eid = jnp.min(jnp.where(sm >= mx, lanes, E), axis=1, keepdims=True)
    eid_ref[...] = eid
    onehot = (lanes == eid).astype(jnp.float32)
    sub = jnp.sum(onehot.reshape(TA // TB, TB, E), axis=1)
    cnt_ref[...] = sub.reshape(TA // TB, 1, E).astype(jnp.int32)


def _tc_gate(x, gate_W):
    return pl.pallas_call(
        _gate_body,
        grid=(GA,),
        in_specs=[
            pl.BlockSpec((TA, D), lambda i: (i, 0)),
            pl.BlockSpec((E, D), lambda i: (0, 0)),
        ],
        out_specs=[
            pl.BlockSpec((TA, E), lambda i: (i, 0)),
            pl.BlockSpec((TA, 1), lambda i: (i, 0)),
            pl.BlockSpec((TA, 1), lambda i: (i, 0)),
            pl.BlockSpec((TA // TB, 1, E), lambda i: (i, 0, 0)),
        ],
        out_shape=[
            jax.ShapeDtypeStruct((T, E), jnp.float32),
            jax.ShapeDtypeStruct((T, 1), jnp.int32),
            jax.ShapeDtypeStruct((T, 1), jnp.float32),
            jax.ShapeDtypeStruct((GB, 1, E), jnp.int32),
        ],
    )(x, gate_W)


def _route_body(eid_ref, cnt_ref, pos_ref, bexp_ref):
    i = pl.program_id(0)
    cnt = cnt_ref[...].reshape(GB, E).astype(jnp.float32)
    rows = lax.broadcasted_iota(jnp.int32, (GB, E), 0)
    pre = jnp.sum(jnp.where(rows < i, cnt, 0.0), axis=0, keepdims=True)
    tot = jnp.sum(cnt, axis=0, keepdims=True)
    nb = jnp.floor((tot + (BT - 1)) / BT)
    nbBT = nb * BT
    er = lax.broadcasted_iota(jnp.int32, (E, E), 0)
    ec = lax.broadcasted_iota(jnp.int32, (E, E), 1)
    nb_col = jnp.broadcast_to(nbBT.reshape(E, 1), (E, E))
    start = jnp.sum(jnp.where(er < ec, nb_col, 0.0), axis=0, keepdims=True)
    base_e = start + pre

    eid = eid_ref[...]
    lanes = lax.broadcasted_iota(jnp.int32, (TB, E), 1)
    oh = (lanes == eid).astype(jnp.float32)
    mr = lax.broadcasted_iota(jnp.int32, (TB, TB), 0)
    mc = lax.broadcasted_iota(jnp.int32, (TB, TB), 1)
    tri = (mc <= mr).astype(jnp.float32)
    csum = lax.dot_general(tri, oh, (((1,), (0,)), ((), ())),
                           preferred_element_type=jnp.float32)
    rank = jnp.sum(oh * csum, axis=1, keepdims=True) - 1.0
    basemap = jnp.sum(oh * base_e, axis=1, keepdims=True)
    pos_ref[...] = (basemap + rank).astype(jnp.int32)

    bs = (start / BT).astype(jnp.int32)
    nbi = nb.astype(jnp.int32)
    ec1 = lax.broadcasted_iota(jnp.int32, (1, E), 1)
    jv = lax.broadcasted_iota(jnp.int32, (1, 1, NBEXP), 2)
    acc = jnp.zeros((1, 1, NBEXP), jnp.int32)
    for e in range(E):
        bs_e = jnp.sum(jnp.where(ec1 == e, bs, 0))
        nb_e = jnp.sum(jnp.where(ec1 == e, nbi, 0))
        inb = jnp.logical_and(jv >= bs_e, jv < bs_e + nb_e)
        acc = jnp.where(inb, e, acc)
    bexp_ref[...] = acc


def _tc_route(eid, cnt3):
    return pl.pallas_call(
        _route_body,
        grid=(GB,),
        in_specs=[
            pl.BlockSpec((TB, 1), lambda i: (i, 0)),
            pl.BlockSpec((GB, 1, E), lambda i: (0, 0, 0)),
        ],
        out_specs=[
            pl.BlockSpec((TB, 1), lambda i: (i, 0)),
            pl.BlockSpec((1, 1, NBEXP), lambda i: (0, 0, 0)),
        ],
        out_shape=[
            jax.ShapeDtypeStruct((T, 1), jnp.int32),
            jax.ShapeDtypeStruct((1, 1, NBEXP), jnp.int32),
        ],
    )(eid, cnt3)


@functools.lru_cache(maxsize=None)
def _sc_mesh():
    return plsc.VectorSubcoreMesh(core_axis_name="c", subcore_axis_name="s",
                                  num_cores=NC, num_subcores=NS)


@functools.lru_cache(maxsize=None)
def _make_sc_scatter():
    @functools.partial(
        pl.kernel,
        out_type=(
            jax.ShapeDtypeStruct((TPAD, D), jnp.float32),
            jax.ShapeDtypeStruct((TPAD,), jnp.float32),
        ),
        mesh=_sc_mesh(),
        scratch_types=[
            pltpu.VMEM((NCS, CS), jnp.int32),
            pltpu.VMEM((NCS, CS), jnp.float32),
            pltpu.VMEM((CS, D), jnp.float32),
            pltpu.VMEM((CS, D), jnp.float32),
            pltpu.SemaphoreType.DMA,
            pltpu.SemaphoreType.DMA,
            pltpu.SemaphoreType.DMA,
        ],
        compiler_params=pltpu.CompilerParams(needs_layout_passes=False),
    )
    def _sc_scatter(x_hbm, pos_hbm, w_hbm, xs_hbm, ws_hbm, posv, wv,
                    row0, row1, lsem, sem, wsem):
        wid = lax.axis_index("s") * NC + lax.axis_index("c")
        base = wid * TPW
        pltpu.sync_copy(pos_hbm.at[wid], posv)
        pltpu.sync_copy(w_hbm.at[wid], wv)
        bufs = [row0, row1]
        lcp = {0: pltpu.async_copy(x_hbm.at[pl.ds(base, CS)], row0, lsem)}
        scp = {}
        for c in range(NCS):
            lcp[c].wait()
            scp[c] = pltpu.async_copy(bufs[c % 2], xs_hbm.at[posv.at[c]], sem)
            nxt = c + 1
            if nxt < NCS:
                if nxt >= 2:
                    scp[nxt - 2].wait()
                lcp[nxt] = pltpu.async_copy(
                    x_hbm.at[pl.ds(base + nxt * CS, CS)], bufs[nxt % 2], lsem)
        wcp = [pltpu.async_copy(wv.at[c], ws_hbm.at[posv.at[c]], wsem)
               for c in range(NCS)]
        for c in range(max(0, NCS - 2), NCS):
            scp[c].wait()
        for cp in wcp:
            cp.wait()

    return _sc_scatter


def _mm_body(bexp_ref, xs_ref, w_ref, b_ref, ws_ref, ys_ref):
    a = xs_ref[...]
    w = w_ref[0]
    mm = lax.dot_general(a, w, (((1,), (1,)), ((), ())),
                         preferred_element_type=jnp.float32)
    ys_ref[...] = (mm + b_ref[0]) * ws_ref[...]


def _tc_mm(bexp, xs, ws, expert_W, expert_b):
    grid_spec = pltpu.PrefetchScalarGridSpec(
        num_scalar_prefetch=1,
        grid=(NBLK,),
        in_specs=[
            pl.BlockSpec((BT, D), lambda i, be: (i, 0)),
            pl.BlockSpec((1, D, D), lambda i, be: (be[i], 0, 0)),
            pl.BlockSpec((1, 1, D), lambda i, be: (be[i], 0, 0)),
            pl.BlockSpec((BT, 1), lambda i, be: (i, 0)),
        ],
        out_specs=pl.BlockSpec((BT, D), lambda i, be: (i, 0)),
    )
    return pl.pallas_call(
        _mm_body,
        grid_spec=grid_spec,
        out_shape=jax.ShapeDtypeStruct((TPAD, D), jnp.float32),
    )(bexp, xs, expert_W, expert_b.reshape(E, 1, D), ws.reshape(TPAD, 1))


@functools.lru_cache(maxsize=None)
def _make_sc_gather():
    @functools.partial(
        pl.kernel,
        out_type=jax.ShapeDtypeStruct((T, D), jnp.float32),
        mesh=_sc_mesh(),
        scratch_types=[
            pltpu.VMEM((NCH, CH), jnp.int32),
            pltpu.VMEM((CH, D), jnp.float32),
            pltpu.SemaphoreType.DMA,
        ],
        compiler_params=pltpu.CompilerParams(needs_layout_passes=False),
    )
    def _sc_gather(ys_hbm, pos_hbm, out_hbm, posv, rowv, sem):
        wid = lax.axis_index("s") * NC + lax.axis_index("c")
        base = wid * TPW
        pltpu.sync_copy(pos_hbm.at[wid], posv)
        for c in range(NCH):
            pltpu.async_copy(ys_hbm.at[posv.at[c]], rowv, sem).wait()
            pltpu.sync_copy(rowv, out_hbm.at[pl.ds(base + c * CH, CH)])

    return _sc_gather


def kernel(x, gate_W, expert_W, expert_b):
    gw, eid, w, cnt3 = _tc_gate(x, gate_W)
    pos, bexp3 = _tc_route(eid, cnt3)
    bexp = bexp3.reshape(NBEXP)
    xs, ws = _make_sc_scatter()(x, pos.reshape(NW, NCS, CS),
                                w.reshape(NW, NCS, CS))
    ys = _tc_mm(bexp, xs, ws, expert_W, expert_b)
    out = _make_sc_gather()(ys, pos.reshape(NW, NCH, CH))
    return out, gw


# speedup vs baseline: 1.2313x; 1.2313x over previous
import functools

import jax
import jax.numpy as jnp
from jax import lax
from jax.experimental import pallas as pl
from jax.experimental.pallas import tpu as pltpu
from jax.experimental.pallas import tpu_sc as plsc

E = 8
D = 768
DA = D + 128
T = 8192
BT = 256
NBLK = T // BT + E
TPAD = NBLK * BT
NBEXP = 48
TA = 1024
GA = T // TA
TB = 256
GB = T // TB
NC, NS = 2, 16
NW = NC * NS
TPW = T // NW
CH = 128
NCH = TPW // CH
CS = 64
NCS = TPW // CS


def _gate_body(x_ref, gw_ref, sm_ref, eid_ref, w_ref, cnt_ref):
    x = x_ref[...]
    logits = lax.dot_general(x, gw_ref[...], (((1,), (1,)), ((), ())),
                             preferred_element_type=jnp.float32)
    m = jnp.max(logits, axis=1, keepdims=True)
    ex = jnp.exp(logits - m)
    sm = ex / jnp.sum(ex, axis=1, keepdims=True)
    sm_ref[...] = sm
    mx = jnp.max(sm, axis=1, keepdims=True)
    w_ref[...] = mx
    lanes = lax.broadcasted_iota(jnp.int32, (TA, E), 1)
    eid = jnp.min(jnp.where(sm >= mx, lanes, E), axis=1, keepdims=True)
    eid_ref[...] = eid
    onehot = (lanes == eid).astype(jnp.float32)
    sub = jnp.sum(onehot.reshape(TA // TB, TB, E), axis=1)
    cnt_ref[...] = sub.reshape(TA // TB, 1, E).astype(jnp.int32)


def _tc_gate(x, gate_W):
    return pl.pallas_call(
        _gate_body,
        grid=(GA,),
        in_specs=[
            pl.BlockSpec((TA, D), lambda i: (i, 0)),
            pl.BlockSpec((E, D), lambda i: (0, 0)),
        ],
        out_specs=[
            pl.BlockSpec((TA, E), lambda i: (i, 0)),
            pl.BlockSpec((TA, 1), lambda i: (i, 0)),
            pl.BlockSpec((TA, 1), lambda i: (i, 0)),
            pl.BlockSpec((TA // TB, 1, E), lambda i: (i, 0, 0)),
        ],
        out_shape=[
            jax.ShapeDtypeStruct((T, E), jnp.float32),
            jax.ShapeDtypeStruct((T, 1), jnp.int32),
            jax.ShapeDtypeStruct((T, 1), jnp.float32),
            jax.ShapeDtypeStruct((GB, 1, E), jnp.int32),
        ],
    )(x, gate_W)


def _route_body(eid_ref, cnt_ref, pos_ref, bexp_ref):
    i = pl.program_id(0)
    cnt = cnt_ref[...].reshape(GB, E).astype(jnp.float32)
    rows = lax.broadcasted_iota(jnp.int32, (GB, E), 0)
    pre = jnp.sum(jnp.where(rows < i, cnt, 0.0), axis=0, keepdims=True)
    tot = jnp.sum(cnt, axis=0, keepdims=True)
    nb = jnp.floor((tot + (BT - 1)) / BT)
    nbBT = nb * BT
    er = lax.broadcasted_iota(jnp.int32, (E, E), 0)
    ec = lax.broadcasted_iota(jnp.int32, (E, E), 1)
    nb_col = jnp.broadcast_to(nbBT.reshape(E, 1), (E, E))
    start = jnp.sum(jnp.where(er < ec, nb_col, 0.0), axis=0, keepdims=True)
    base_e = start + pre

    eid = eid_ref[...]
    lanes = lax.broadcasted_iota(jnp.int32, (TB, E), 1)
    oh = (lanes == eid).astype(jnp.float32)
    mr = lax.broadcasted_iota(jnp.int32, (TB, TB), 0)
    mc = lax.broadcasted_iota(jnp.int32, (TB, TB), 1)
    tri = (mc <= mr).astype(jnp.float32)
    csum = lax.dot_general(tri, oh, (((1,), (0,)), ((), ())),
                           preferred_element_type=jnp.float32)
    rank = jnp.sum(oh * csum, axis=1, keepdims=True) - 1.0
    basemap = jnp.sum(oh * base_e, axis=1, keepdims=True)
    pos_ref[...] = (basemap + rank).astype(jnp.int32)

    bs = (start / BT).astype(jnp.int32)
    nbi = nb.astype(jnp.int32)
    ec1 = lax.broadcasted_iota(jnp.int32, (1, E), 1)
    jv = lax.broadcasted_iota(jnp.int32, (1, 1, NBEXP), 2)
    acc = jnp.zeros((1, 1, NBEXP), jnp.int32)
    for e in range(E):
        bs_e = jnp.sum(jnp.where(ec1 == e, bs, 0))
        nb_e = jnp.sum(jnp.where(ec1 == e, nbi, 0))
        inb = jnp.logical_and(jv >= bs_e, jv < bs_e + nb_e)
        acc = jnp.where(inb, e, acc)
    bexp_ref[...] = acc


def _tc_route(eid, cnt3):
    return pl.pallas_call(
        _route_body,
        grid=(GB,),
        in_specs=[
            pl.BlockSpec((TB, 1), lambda i: (i, 0)),
            pl.BlockSpec((GB, 1, E), lambda i: (0, 0, 0)),
        ],
        out_specs=[
            pl.BlockSpec((TB, 1), lambda i: (i, 0)),
            pl.BlockSpec((1, 1, NBEXP), lambda i: (0, 0, 0)),
        ],
        out_shape=[
            jax.ShapeDtypeStruct((T, 1), jnp.int32),
            jax.ShapeDtypeStruct((1, 1, NBEXP), jnp.int32),
        ],
    )(eid, cnt3)


@functools.lru_cache(maxsize=None)
def _sc_mesh():
    return plsc.VectorSubcoreMesh(core_axis_name="c", subcore_axis_name="s",
                                  num_cores=NC, num_subcores=NS)


@functools.lru_cache(maxsize=None)
def _make_sc_scatter():
    @functools.partial(
        pl.kernel,
        out_type=jax.ShapeDtypeStruct((TPAD, DA), jnp.float32),
        mesh=_sc_mesh(),
        scratch_types=[
            pltpu.VMEM((NCS, CS), jnp.int32),
            pltpu.VMEM((NCS, CS), jnp.float32),
            pltpu.VMEM((CS, DA), jnp.float32),
            pltpu.VMEM((CS, DA), jnp.float32),
            pltpu.SemaphoreType.DMA,
            pltpu.SemaphoreType.DMA,
        ],
        compiler_params=pltpu.CompilerParams(needs_layout_passes=False),
    )
    def _sc_scatter(x_hbm, pos_hbm, w_hbm, xs_hbm, posv, wv,
                    row0, row1, lsem, sem):
        wid = lax.axis_index("s") * NC + lax.axis_index("c")
        base = wid * TPW
        pltpu.sync_copy(pos_hbm.at[wid], posv)
        pltpu.sync_copy(w_hbm.at[wid], wv)
        lanes = lax.iota(jnp.int32, 16)
        colD = jnp.full((16,), D, jnp.int32)
        bufs = [row0, row1]
        lcp = {0: pltpu.async_copy(x_hbm.at[pl.ds(base, CS)],
                                   row0.at[:, pl.ds(0, D)], lsem)}
        scp = {}
        for c in range(NCS):
            lcp[c].wait()
            for g in range(CS // 16):
                plsc.store_scatter(bufs[c % 2],
                                   [g * 16 + lanes, colD],
                                   wv[c, pl.ds(g * 16, 16)])
            scp[c] = pltpu.async_copy(bufs[c % 2], xs_hbm.at[posv.at[c]], sem)
            nxt = c + 1
            if nxt < NCS:
                if nxt >= 2:
                    scp[nxt - 2].wait()
                lcp[nxt] = pltpu.async_copy(
                    x_hbm.at[pl.ds(base + nxt * CS, CS)],
                    bufs[nxt % 2].at[:, pl.ds(0, D)], lsem)
        for c in range(max(0, NCS - 2), NCS):
            scp[c].wait()

    return _sc_scatter


def _mm_body(bexp_ref, xs_ref, w_ref, b_ref, ys_ref):
    blk = xs_ref[...]
    a = blk[:, :D]
    scale = blk[:, D:D + 1]
    w = w_ref[0]
    mm = lax.dot_general(a, w, (((1,), (1,)), ((), ())),
                         preferred_element_type=jnp.float32)
    ys_ref[...] = (mm + b_ref[0]) * scale


def _tc_mm(bexp, xs, expert_W, expert_b):
    grid_spec = pltpu.PrefetchScalarGridSpec(
        num_scalar_prefetch=1,
        grid=(NBLK,),
        in_specs=[
            pl.BlockSpec((BT, DA), lambda i, be: (i, 0)),
            pl.BlockSpec((1, D, D), lambda i, be: (be[i], 0, 0)),
            pl.BlockSpec((1, 1, D), lambda i, be: (be[i], 0, 0)),
        ],
        out_specs=pl.BlockSpec((BT, D), lambda i, be: (i, 0)),
    )
    return pl.pallas_call(
        _mm_body,
        grid_spec=grid_spec,
        out_shape=jax.ShapeDtypeStruct((TPAD, D), jnp.float32),
    )(bexp, xs, expert_W, expert_b.reshape(E, 1, D))


@functools.lru_cache(maxsize=None)
def _make_sc_gather():
    @functools.partial(
        pl.kernel,
        out_type=jax.ShapeDtypeStruct((T, D), jnp.float32),
        mesh=_sc_mesh(),
        scratch_types=[
            pltpu.VMEM((NCH, CH), jnp.int32),
            pltpu.VMEM((CH, D), jnp.float32),
            pltpu.SemaphoreType.DMA,
        ],
        compiler_params=pltpu.CompilerParams(needs_layout_passes=False),
    )
    def _sc_gather(ys_hbm, pos_hbm, out_hbm, posv, rowv, sem):
        wid = lax.axis_index("s") * NC + lax.axis_index("c")
        base = wid * TPW
        pltpu.sync_copy(pos_hbm.at[wid], posv)
        for c in range(NCH):
            pltpu.async_copy(ys_hbm.at[posv.at[c]], rowv, sem).wait()
            pltpu.sync_copy(rowv, out_hbm.at[pl.ds(base + c * CH, CH)])

    return _sc_gather


def kernel(x, gate_W, expert_W, expert_b):
    gw, eid, w, cnt3 = _tc_gate(x, gate_W)
    pos, bexp3 = _tc_route(eid, cnt3)
    bexp = bexp3.reshape(NBEXP)
    xs = _make_sc_scatter()(x, pos.reshape(NW, NCS, CS),
                            w.reshape(NW, NCS, CS))
    ys = _tc_mm(bexp, xs, expert_W, expert_b)
    out = _make_sc_gather()(ys, pos.reshape(NW, NCH, CH))
    return out, gw
